# Initial kernel scaffold; baseline (speedup 1.0000x reference)
import jax
import jax.numpy as jnp
from jax.experimental import pallas as pl


def kernel(inputs, table0, table1):
    raise NotImplementedError("write your pallas kernel here")


# speedup vs baseline: 1.4789x; 1.4789x over previous
import functools

import jax
import jax.numpy as jnp
from jax import lax
from jax.experimental import pallas as pl
from jax.experimental.pallas import tpu as pltpu
from jax.experimental.pallas import tpu_sc as plsc

NC, NS = 2, 16
NW = NC * NS
GRP = 128
K = 8
D0, D1 = 32, 16
D = D0 + D1


def _sc_body(idx_hbm, t0_hbm, t1_hbm, out_hbm, idx_v, rows0, rows1, sem0, sem1):
    wid = lax.axis_index("s") * NC + lax.axis_index("c")
    groups_total = idx_hbm.shape[0]
    gpw = groups_total // NW
    chunks = gpw // K
    base_g = wid * gpw

    @pl.loop(0, chunks)
    def _chunk(c):
        g0 = base_g + c * K
        pltpu.sync_copy(idx_hbm.at[pl.ds(g0, K)], idx_v)
        cps = []
        for j in range(K):
            cps.append(pltpu.async_copy(t0_hbm.at[idx_v.at[j]], rows0.at[j], sem0))
            cps.append(pltpu.async_copy(t1_hbm.at[idx_v.at[j]], rows1.at[j], sem1))
        for cp in cps:
            cp.wait()
        row0 = g0 * GRP
        for j in range(K):
            pltpu.sync_copy(rows0.at[j],
                            out_hbm.at[pl.ds(row0 + j * GRP, GRP), pl.ds(0, D0)])
            pltpu.sync_copy(rows1.at[j],
                            out_hbm.at[pl.ds(row0 + j * GRP, GRP), pl.ds(D0, D1)])


def kernel(inputs, table0, table1):
    B, L = inputs.shape
    n = B * L
    idx = inputs.reshape(n // GRP, GRP).astype(jnp.int32)
    mesh = plsc.VectorSubcoreMesh(core_axis_name="c", subcore_axis_name="s")
    out = pl.kernel(
        functools.partial(_sc_body),
        out_type=jax.ShapeDtypeStruct((n, D), jnp.float32),
        mesh=mesh,
        compiler_params=pltpu.CompilerParams(use_tc_tiling_on_sc=False),
        scratch_types=[
            pltpu.VMEM((K, GRP), jnp.int32),
            pltpu.VMEM((K, GRP, D0), jnp.float32),
            pltpu.VMEM((K, GRP, D1), jnp.float32),
            pltpu.SemaphoreType.DMA,
            pltpu.SemaphoreType.DMA,
        ],
    )(idx, table0, table1)
    return out.reshape(B, L, D)


# speedup vs baseline: 1.4983x; 1.0131x over previous
import jax
import jax.numpy as jnp
from jax import lax
from jax.experimental import pallas as pl
from jax.experimental.pallas import tpu as pltpu
from jax.experimental.pallas import tpu_sc as plsc

NC, NS = 2, 16
NW = NC * NS
GRP = 128
K = 8
CHUNK = K * GRP
D0, D1 = 32, 16
D = D0 + D1


def _sc_body(idx_hbm, t0_hbm, t1_hbm, out_hbm, idx_v, rows0, rows1, sem0, sem1):
    wid = lax.axis_index("s") * NC + lax.axis_index("c")
    groups_total = idx_hbm.shape[0] // GRP
    gpw = groups_total // NW
    chunks = gpw // K
    base_g = wid * gpw

    @pl.loop(0, chunks)
    def _chunk(c):
        g0 = base_g + c * K
        pltpu.sync_copy(idx_hbm.at[pl.ds(g0 * GRP, CHUNK)], idx_v)
        cp0 = pltpu.async_copy(t0_hbm.at[idx_v], rows0, sem0)
        cp1 = pltpu.async_copy(t1_hbm.at[idx_v], rows1, sem1)
        cp0.wait()
        cp1.wait()
        row0 = g0 * GRP
        pltpu.sync_copy(rows0, out_hbm.at[pl.ds(row0, CHUNK), pl.ds(0, D0)])
        pltpu.sync_copy(rows1, out_hbm.at[pl.ds(row0, CHUNK), pl.ds(D0, D1)])


def kernel(inputs, table0, table1):
    B, L = inputs.shape
    n = B * L
    idx = inputs.reshape(n).astype(jnp.int32)
    mesh = plsc.VectorSubcoreMesh(core_axis_name="c", subcore_axis_name="s")
    out = pl.kernel(
        _sc_body,
        out_type=jax.ShapeDtypeStruct((n, D), jnp.float32),
        mesh=mesh,
        compiler_params=pltpu.CompilerParams(use_tc_tiling_on_sc=False),
        scratch_types=[
            pltpu.VMEM((CHUNK,), jnp.int32),
            pltpu.VMEM((CHUNK, D0), jnp.float32),
            pltpu.VMEM((CHUNK, D1), jnp.float32),
            pltpu.SemaphoreType.DMA,
            pltpu.SemaphoreType.DMA,
        ],
    )(idx, table0, table1)
    return out.reshape(B, L, D)


# speedup vs baseline: 1.5355x; 1.0249x over previous
import jax
import jax.numpy as jnp
from jax import lax
from jax.experimental import pallas as pl
from jax.experimental.pallas import tpu as pltpu
from jax.experimental.pallas import tpu_sc as plsc

NC, NS = 2, 16
NW = NC * NS
CHUNK = 512
D0, D1 = 32, 16
D = D0 + D1


def _sc_body(idx_hbm, t0_hbm, t1_hbm, out_hbm,
             idx_v, rows0, rows1, gsem0, gsem1, wsem0, wsem1):
    wid = lax.axis_index("s") * NC + lax.axis_index("c")
    n = idx_hbm.shape[0]
    npw = n // NW
    chunks = npw // CHUNK
    base = wid * npw

    pltpu.sync_copy(idx_hbm.at[pl.ds(base, npw)], idx_v)

    gsems = (gsem0, gsem1)

    def fire_gathers(i, b):
        src = idx_v.at[pl.ds(i * CHUNK, CHUNK)]
        pltpu.async_copy(t0_hbm.at[src], rows0.at[b], gsems[b])
        pltpu.async_copy(t1_hbm.at[src], rows1.at[b], gsems[b])

    def wait_gathers(b):
        pltpu.make_async_copy(t0_hbm.at[idx_v.at[pl.ds(0, CHUNK)]],
                              rows0.at[b], gsems[b]).wait()
        pltpu.make_async_copy(t1_hbm.at[idx_v.at[pl.ds(0, CHUNK)]],
                              rows1.at[b], gsems[b]).wait()

    def fire_writes(i, b, wsem):
        row = base + i * CHUNK
        pltpu.async_copy(rows0.at[b],
                         out_hbm.at[pl.ds(row, CHUNK), pl.ds(0, D0)], wsem)
        pltpu.async_copy(rows1.at[b],
                         out_hbm.at[pl.ds(row, CHUNK), pl.ds(D0, D1)], wsem)

    def drain_writes(b, wsem):
        pltpu.make_async_copy(rows0.at[b],
                              out_hbm.at[pl.ds(base, CHUNK), pl.ds(0, D0)],
                              wsem).wait()
        pltpu.make_async_copy(rows1.at[b],
                              out_hbm.at[pl.ds(base, CHUNK), pl.ds(D0, D1)],
                              wsem).wait()

    fire_gathers(0, 0)
    fire_gathers(1, 1)
    wait_gathers(0)
    fire_writes(0, 0, wsem0)

    @pl.loop(1, chunks - 3, step=2)
    def _pair(i):
        drain_writes(0, wsem0)
        fire_gathers(i + 1, 0)
        wait_gathers(1)
        fire_writes(i, 1, wsem1)
        drain_writes(1, wsem1)
        fire_gathers(i + 2, 1)
        wait_gathers(0)
        fire_writes(i + 1, 0, wsem0)

    i = chunks - 3
    drain_writes(0, wsem0)
    fire_gathers(i + 1, 0)
    wait_gathers(1)
    fire_writes(i, 1, wsem1)

    drain_writes(1, wsem1)
    fire_gathers(i + 2, 1)
    wait_gathers(0)
    fire_writes(i + 1, 0, wsem0)

    wait_gathers(1)
    fire_writes(i + 2, 1, wsem1)

    drain_writes(0, wsem0)
    drain_writes(1, wsem1)


def kernel(inputs, table0, table1):
    B, L = inputs.shape
    n = B * L
    idx = inputs.reshape(n).astype(jnp.int32)
    mesh = plsc.VectorSubcoreMesh(core_axis_name="c", subcore_axis_name="s")
    out = pl.kernel(
        _sc_body,
        out_type=jax.ShapeDtypeStruct((n, D), jnp.float32),
        mesh=mesh,
        compiler_params=pltpu.CompilerParams(use_tc_tiling_on_sc=False),
        scratch_types=[
            pltpu.VMEM((n // NW,), jnp.int32),
            pltpu.VMEM((2, CHUNK, D0), jnp.float32),
            pltpu.VMEM((2, CHUNK, D1), jnp.float32),
            pltpu.SemaphoreType.DMA,
            pltpu.SemaphoreType.DMA,
            pltpu.SemaphoreType.DMA,
            pltpu.SemaphoreType.DMA,
        ],
    )(idx, table0, table1)
    return out.reshape(B, L, D)
